# baseline (speedup 1.0000x reference)
import functools
import math

import jax
import jax.numpy as jnp
from jax import lax
from jax.experimental import pallas as pl
from jax.experimental.pallas import tpu as pltpu
from jax.experimental.pallas import tpu_sc as plsc

B, H, LQ, LK = 1, 16, 2048, 2048
D_K = 128
D_V = 128
D_MODEL = 2048

N_ROWS = H * LQ

SC_ROWS = 4
NW = 32
RPW = N_ROWS // NW
NCH = RPW // SC_ROWS
NLANE = 16
NBUF = 2


def _sc_combine(gab2, bias2, iud, wu_r, wd_r):
    mesh = plsc.VectorSubcoreMesh(core_axis_name="core", subcore_axis_name="subcore")

    @functools.partial(
        pl.kernel,
        out_type=jax.ShapeDtypeStruct((N_ROWS, LK), jnp.float32),
        mesh=mesh,
        scratch_types=[
            pltpu.VMEM((RPW * 2,), jnp.int32),
            pltpu.VMEM((RPW * NLANE,), jnp.float32),
            pltpu.VMEM((RPW * NLANE,), jnp.float32),
            pltpu.VMEM((NBUF, 2 * SC_ROWS, LK), jnp.float32),
            pltpu.VMEM((NBUF, SC_ROWS, LK), jnp.float32),
            pltpu.VMEM((NBUF, SC_ROWS, LK), jnp.float32),
            pltpu.SemaphoreType.DMA,
            pltpu.SemaphoreType.DMA,
            pltpu.SemaphoreType.DMA,
            pltpu.SemaphoreType.DMA,
            pltpu.SemaphoreType.DMA,
            pltpu.SemaphoreType.DMA,
        ],
    )
    def run(gab_hbm, bias_hbm, iud_hbm, wu_hbm, wd_hbm, out_hbm,
            iud_v, wu_v, wd_v, ud_v, b_v, o_v,
            sem_g0, sem_g1, sem_b0, sem_b1, sem_o0, sem_o1):
        cid = lax.axis_index("core")
        sid = lax.axis_index("subcore")
        wid = sid * 2 + cid
        base = wid * RPW
        pltpu.sync_copy(iud_hbm.at[pl.ds(base * 2, RPW * 2)], iud_v)
        pltpu.sync_copy(wu_hbm.at[pl.ds(base * NLANE, RPW * NLANE)], wu_v)
        pltpu.sync_copy(wd_hbm.at[pl.ds(base * NLANE, RPW * NLANE)], wd_v)

        sem_g = (sem_g0, sem_g1)
        sem_b = (sem_b0, sem_b1)
        sem_o = (sem_o0, sem_o1)

        def issue_in(c, b):
            pltpu.async_copy(
                gab_hbm.at[iud_v.at[pl.ds(c * (2 * SC_ROWS), 2 * SC_ROWS)]],
                ud_v.at[b], sem_g[b])
            pltpu.async_copy(
                bias_hbm.at[pl.ds(base + c * SC_ROWS, SC_ROWS)], b_v.at[b], sem_b[b])

        issue_in(0, 0)
        issue_in(1, 1)

        @pl.loop(0, NCH, step=NBUF)
        def _(c0):
            for b in range(NBUF):
                cur = c0 + b

                @pl.when(cur >= NBUF)
                def _(b=b):
                    pltpu.make_async_copy(
                        o_v.at[b], out_hbm.at[pl.ds(base, SC_ROWS)], sem_o[b]).wait()

                pltpu.make_async_copy(
                    gab_hbm.at[iud_v.at[pl.ds(cur * (2 * SC_ROWS), 2 * SC_ROWS)]],
                    ud_v.at[b], sem_g[b]).wait()
                pltpu.make_async_copy(
                    bias_hbm.at[pl.ds(base, SC_ROWS)], b_v.at[b], sem_b[b]).wait()

                for r in range(SC_ROWS):
                    wsl = pl.ds((cur * SC_ROWS + r) * NLANE, NLANE)
                    wu_vec = wu_v[wsl]
                    wd_vec = wd_v[wsl]

                    @pl.loop(0, LK, step=64)
                    def _(col, r=r, b=b, wu_vec=wu_vec, wd_vec=wd_vec):
                        for cc in range(0, 64, NLANE):
                            sl = pl.ds(col + cc, NLANE)
                            g = (ud_v[b, r, sl] * wu_vec
                                 + ud_v[b, SC_ROWS + r, sl] * wd_vec)
                            o_v[b, r, sl] = b_v[b, r, sl] - g * g

                pltpu.async_copy(
                    o_v.at[b], out_hbm.at[pl.ds(base + cur * SC_ROWS, SC_ROWS)],
                    sem_o[b])

                @pl.when(cur + NBUF < NCH)
                def _(cur=cur, b=b):
                    issue_in(cur + NBUF, b)

        for b in range(NBUF):
            pltpu.make_async_copy(
                o_v.at[b], out_hbm.at[pl.ds(base, SC_ROWS)], sem_o[b]).wait()

    return run(gab2, bias2, iud, wu_r, wd_r)


BQ, BK = 256, 256
NQ, NK = LQ // BQ, LK // BK


def _attn_body(q_ref, k_ref, v_ref, c_ref, w_ref, g_ref, acc_ref):
    ik = pl.program_id(1)
    scale = 1.0 / math.sqrt(D_K)
    m = None
    for h in range(H):
        a = lax.dot_general(
            q_ref[h] * scale, k_ref[h], (((1,), (1,)), ((), ())),
            preferred_element_type=jnp.float32)
        a = a + c_ref[h]
        w_ref[h] = a
        m = a if h == 0 else jnp.maximum(m, a)
    s = None
    for h in range(H):
        e = jnp.exp(w_ref[h] - m)
        w_ref[h] = e
        s = e if h == 0 else s + e
    r = 1.0 / s
    for h in range(H):
        wgt = w_ref[h] * r
        w_ref[h] = wgt
        pv = lax.dot_general(
            wgt, v_ref[h], (((1,), (0,)), ((), ())),
            preferred_element_type=jnp.float32)

        @pl.when(ik == 0)
        def _(pv=pv, h=h):
            acc_ref[h] = pv

        @pl.when(ik > 0)
        def _(pv=pv, h=h):
            acc_ref[h] = acc_ref[h] + pv

    @pl.when(ik == NK - 1)
    def _():
        for h in range(H):
            g_ref[h] = acc_ref[h]


def _attention(q3, k3, v3, comb3):
    return pl.pallas_call(
        _attn_body,
        grid=(NQ, NK),
        in_specs=[
            pl.BlockSpec((H, BQ, D_K), lambda iq, ik: (0, iq, 0)),
            pl.BlockSpec((H, BK, D_K), lambda iq, ik: (0, ik, 0)),
            pl.BlockSpec((H, BK, D_V), lambda iq, ik: (0, ik, 0)),
            pl.BlockSpec((H, BQ, BK), lambda iq, ik: (0, iq, ik)),
        ],
        out_specs=[
            pl.BlockSpec((H, BQ, BK), lambda iq, ik: (0, iq, ik)),
            pl.BlockSpec((H, BQ, D_V), lambda iq, ik: (0, iq, 0)),
        ],
        out_shape=[
            jax.ShapeDtypeStruct((H, LQ, LK), jnp.float32),
            jax.ShapeDtypeStruct((H, LQ, D_V), jnp.float32),
        ],
        scratch_shapes=[pltpu.VMEM((H, BQ, D_V), jnp.float32)],
        compiler_params=pltpu.CompilerParams(
            dimension_semantics=("parallel", "arbitrary")),
    )(q3, k3, v3, comb3)


BQ2 = 256
NQ2 = LQ // BQ2


def _proj_body(x_ref, w_ref, b_ref, o_ref):
    acc = None
    for h in range(H):
        pv = lax.dot_general(
            x_ref[h], w_ref[h * D_V:(h + 1) * D_V, :], (((1,), (0,)), ((), ())),
            preferred_element_type=jnp.float32)
        acc = pv if h == 0 else acc + pv
    o_ref[...] = acc + b_ref[0][None, :]


def _proj(gout, W_out, b_out2):
    return pl.pallas_call(
        _proj_body,
        grid=(NQ2,),
        in_specs=[
            pl.BlockSpec((H, BQ2, D_V), lambda i: (0, i, 0)),
            pl.BlockSpec((D_MODEL, D_MODEL), lambda i: (0, 0)),
            pl.BlockSpec((1, D_MODEL), lambda i: (0, 0)),
        ],
        out_specs=pl.BlockSpec((BQ2, D_MODEL), lambda i: (i, 0)),
        out_shape=jax.ShapeDtypeStruct((LQ, D_MODEL), jnp.float32),
        compiler_params=pltpu.CompilerParams(
            dimension_semantics=("arbitrary",)),
    )(gout, W_out, b_out2)


def kernel(q, k, v, bias, graph_attn_bias, W_pos_v, b_pos_v, W_pos_s, b_pos_s,
           W_out, b_out):
    q3 = q.reshape(H, LQ, D_K)
    k3 = k.reshape(H, LK, D_K)
    v3 = v.reshape(H, LK, D_V)
    gab2 = graph_attn_bias.reshape(H * LK, LK)
    bias2 = bias.reshape(N_ROWS, LK)

    scaled_q = q / (D_K ** 0.5)
    pos_v = scaled_q @ W_pos_v + b_pos_v
    pos_s = jnp.tanh(pos_v) @ W_pos_s + b_pos_s
    pos = jax.nn.sigmoid(pos_s) * (LK - 1)
    pos_up = jnp.ceil(pos).astype(jnp.int32)
    pos_down = jnp.floor(pos).astype(jnp.int32)
    wu = 1.0 - (pos_up.astype(jnp.float32) - pos)
    wd = 1.0 - (pos - pos_down.astype(jnp.float32))

    row_base = (jnp.arange(H, dtype=jnp.int32) * LK).reshape(1, H, 1, 1)
    iu = (pos_up + row_base).reshape(N_ROWS)
    idn = (pos_down + row_base).reshape(N_ROWS)
    iud = jnp.stack(
        [iu.reshape(-1, SC_ROWS), idn.reshape(-1, SC_ROWS)], axis=1).reshape(-1)
    s2 = math.sqrt(0.5)
    wu_r = jnp.broadcast_to((wu * s2).reshape(N_ROWS, 1), (N_ROWS, NLANE)).reshape(-1)
    wd_r = jnp.broadcast_to((wd * s2).reshape(N_ROWS, 1), (N_ROWS, NLANE)).reshape(-1)

    comb3 = _sc_combine(gab2, bias2, iud, wu_r, wd_r).reshape(H, LQ, LK)
    weights3, gout = _attention(q3, k3, v3, comb3)
    out2 = _proj(gout, W_out, b_out.reshape(1, D_MODEL))
    return out2.reshape(B, LQ, D_MODEL), weights3.reshape(B, H, LQ, LK)


# speedup vs baseline: 1.8226x; 1.8226x over previous
import functools
import math

import jax
import jax.numpy as jnp
from jax import lax
from jax.experimental import pallas as pl
from jax.experimental.pallas import tpu as pltpu
from jax.experimental.pallas import tpu_sc as plsc

B, H, LQ, LK = 1, 16, 2048, 2048
D_K = 128
D_V = 128
D_MODEL = 2048

N_ROWS = H * LQ

SC_ROWS = 4
NW = 32
RPW = N_ROWS // NW
NCH = RPW // SC_ROWS
NLANE = 16
NBUF = 2


def _sc_combine(gab2, bias2, iud, wu_r, wd_r):
    mesh = plsc.VectorSubcoreMesh(core_axis_name="core", subcore_axis_name="subcore")

    @functools.partial(
        pl.kernel,
        out_type=jax.ShapeDtypeStruct((N_ROWS, LK), jnp.float32),
        mesh=mesh,
        scratch_types=[
            pltpu.VMEM((RPW * 2,), jnp.int32),
            pltpu.VMEM((RPW * NLANE,), jnp.float32),
            pltpu.VMEM((RPW * NLANE,), jnp.float32),
            pltpu.VMEM((NBUF, 2 * SC_ROWS, LK), jnp.float32),
            pltpu.VMEM((NBUF, SC_ROWS, LK), jnp.float32),
            pltpu.VMEM((NBUF, SC_ROWS, LK), jnp.float32),
            pltpu.SemaphoreType.DMA,
            pltpu.SemaphoreType.DMA,
            pltpu.SemaphoreType.DMA,
            pltpu.SemaphoreType.DMA,
            pltpu.SemaphoreType.DMA,
            pltpu.SemaphoreType.DMA,
        ],
    )
    def run(gab_hbm, bias_hbm, iud_hbm, wu_hbm, wd_hbm, out_hbm,
            iud_v, wu_v, wd_v, ud_v, b_v, o_v,
            sem_g0, sem_g1, sem_b0, sem_b1, sem_o0, sem_o1):
        cid = lax.axis_index("core")
        sid = lax.axis_index("subcore")
        wid = sid * 2 + cid
        base = wid * RPW
        pltpu.sync_copy(iud_hbm.at[pl.ds(base * 2, RPW * 2)], iud_v)
        pltpu.sync_copy(wu_hbm.at[pl.ds(base * NLANE, RPW * NLANE)], wu_v)
        pltpu.sync_copy(wd_hbm.at[pl.ds(base * NLANE, RPW * NLANE)], wd_v)

        sem_g = (sem_g0, sem_g1)
        sem_b = (sem_b0, sem_b1)
        sem_o = (sem_o0, sem_o1)

        def issue_in(c, b):
            pltpu.async_copy(
                gab_hbm.at[iud_v.at[pl.ds(c * (2 * SC_ROWS), 2 * SC_ROWS)]],
                ud_v.at[b], sem_g[b])
            pltpu.async_copy(
                bias_hbm.at[pl.ds(base + c * SC_ROWS, SC_ROWS)], b_v.at[b], sem_b[b])

        issue_in(0, 0)
        issue_in(1, 1)

        @pl.loop(0, NCH, step=NBUF)
        def _(c0):
            for b in range(NBUF):
                cur = c0 + b

                @pl.when(cur >= NBUF)
                def _(b=b):
                    pltpu.make_async_copy(
                        o_v.at[b], out_hbm.at[pl.ds(base, SC_ROWS)], sem_o[b]).wait()

                pltpu.make_async_copy(
                    gab_hbm.at[iud_v.at[pl.ds(cur * (2 * SC_ROWS), 2 * SC_ROWS)]],
                    ud_v.at[b], sem_g[b]).wait()
                pltpu.make_async_copy(
                    bias_hbm.at[pl.ds(base, SC_ROWS)], b_v.at[b], sem_b[b]).wait()

                for r in range(SC_ROWS):
                    wsl = pl.ds((cur * SC_ROWS + r) * NLANE, NLANE)
                    wu_vec = wu_v[wsl]
                    wd_vec = wd_v[wsl]

                    @plsc.parallel_loop(0, LK, NLANE, unroll=8)
                    def _(col, r=r, b=b, wu_vec=wu_vec, wd_vec=wd_vec):
                        sl = pl.ds(col, NLANE)
                        g = (ud_v[b, r, sl] * wu_vec
                             + ud_v[b, SC_ROWS + r, sl] * wd_vec)
                        o_v[b, r, sl] = b_v[b, r, sl] - g * g

                pltpu.async_copy(
                    o_v.at[b], out_hbm.at[pl.ds(base + cur * SC_ROWS, SC_ROWS)],
                    sem_o[b])

                @pl.when(cur + NBUF < NCH)
                def _(cur=cur, b=b):
                    issue_in(cur + NBUF, b)

        for b in range(NBUF):
            pltpu.make_async_copy(
                o_v.at[b], out_hbm.at[pl.ds(base, SC_ROWS)], sem_o[b]).wait()

    return run(gab2, bias2, iud, wu_r, wd_r)


BQ, BK = 256, 256
NQ, NK = LQ // BQ, LK // BK


def _attn_body(q_ref, k_ref, v_ref, c_ref, w_ref, g_ref, acc_ref):
    ik = pl.program_id(1)
    scale = 1.0 / math.sqrt(D_K)
    m = None
    for h in range(H):
        a = lax.dot_general(
            q_ref[h] * scale, k_ref[h], (((1,), (1,)), ((), ())),
            preferred_element_type=jnp.float32)
        a = a + c_ref[h]
        w_ref[h] = a
        m = a if h == 0 else jnp.maximum(m, a)
    s = None
    for h in range(H):
        e = jnp.exp(w_ref[h] - m)
        w_ref[h] = e
        s = e if h == 0 else s + e
    r = 1.0 / s
    for h in range(H):
        wgt = w_ref[h] * r
        w_ref[h] = wgt
        pv = lax.dot_general(
            wgt, v_ref[h], (((1,), (0,)), ((), ())),
            preferred_element_type=jnp.float32)

        @pl.when(ik == 0)
        def _(pv=pv, h=h):
            acc_ref[h] = pv

        @pl.when(ik > 0)
        def _(pv=pv, h=h):
            acc_ref[h] = acc_ref[h] + pv

    @pl.when(ik == NK - 1)
    def _():
        for h in range(H):
            g_ref[h] = acc_ref[h]


def _attention(q3, k3, v3, comb3):
    return pl.pallas_call(
        _attn_body,
        grid=(NQ, NK),
        in_specs=[
            pl.BlockSpec((H, BQ, D_K), lambda iq, ik: (0, iq, 0)),
            pl.BlockSpec((H, BK, D_K), lambda iq, ik: (0, ik, 0)),
            pl.BlockSpec((H, BK, D_V), lambda iq, ik: (0, ik, 0)),
            pl.BlockSpec((H, BQ, BK), lambda iq, ik: (0, iq, ik)),
        ],
        out_specs=[
            pl.BlockSpec((H, BQ, BK), lambda iq, ik: (0, iq, ik)),
            pl.BlockSpec((H, BQ, D_V), lambda iq, ik: (0, iq, 0)),
        ],
        out_shape=[
            jax.ShapeDtypeStruct((H, LQ, LK), jnp.float32),
            jax.ShapeDtypeStruct((H, LQ, D_V), jnp.float32),
        ],
        scratch_shapes=[pltpu.VMEM((H, BQ, D_V), jnp.float32)],
        compiler_params=pltpu.CompilerParams(
            dimension_semantics=("parallel", "arbitrary")),
    )(q3, k3, v3, comb3)


BQ2 = 256
NQ2 = LQ // BQ2


def _proj_body(x_ref, w_ref, b_ref, o_ref):
    acc = None
    for h in range(H):
        pv = lax.dot_general(
            x_ref[h], w_ref[h * D_V:(h + 1) * D_V, :], (((1,), (0,)), ((), ())),
            preferred_element_type=jnp.float32)
        acc = pv if h == 0 else acc + pv
    o_ref[...] = acc + b_ref[0][None, :]


def _proj(gout, W_out, b_out2):
    return pl.pallas_call(
        _proj_body,
        grid=(NQ2,),
        in_specs=[
            pl.BlockSpec((H, BQ2, D_V), lambda i: (0, i, 0)),
            pl.BlockSpec((D_MODEL, D_MODEL), lambda i: (0, 0)),
            pl.BlockSpec((1, D_MODEL), lambda i: (0, 0)),
        ],
        out_specs=pl.BlockSpec((BQ2, D_MODEL), lambda i: (i, 0)),
        out_shape=jax.ShapeDtypeStruct((LQ, D_MODEL), jnp.float32),
        compiler_params=pltpu.CompilerParams(
            dimension_semantics=("arbitrary",)),
    )(gout, W_out, b_out2)


def kernel(q, k, v, bias, graph_attn_bias, W_pos_v, b_pos_v, W_pos_s, b_pos_s,
           W_out, b_out):
    q3 = q.reshape(H, LQ, D_K)
    k3 = k.reshape(H, LK, D_K)
    v3 = v.reshape(H, LK, D_V)
    gab2 = graph_attn_bias.reshape(H * LK, LK)
    bias2 = bias.reshape(N_ROWS, LK)

    scaled_q = q / (D_K ** 0.5)
    pos_v = scaled_q @ W_pos_v + b_pos_v
    pos_s = jnp.tanh(pos_v) @ W_pos_s + b_pos_s
    pos = jax.nn.sigmoid(pos_s) * (LK - 1)
    pos_up = jnp.ceil(pos).astype(jnp.int32)
    pos_down = jnp.floor(pos).astype(jnp.int32)
    wu = 1.0 - (pos_up.astype(jnp.float32) - pos)
    wd = 1.0 - (pos - pos_down.astype(jnp.float32))

    row_base = (jnp.arange(H, dtype=jnp.int32) * LK).reshape(1, H, 1, 1)
    iu = (pos_up + row_base).reshape(N_ROWS)
    idn = (pos_down + row_base).reshape(N_ROWS)
    iud = jnp.stack(
        [iu.reshape(-1, SC_ROWS), idn.reshape(-1, SC_ROWS)], axis=1).reshape(-1)
    s2 = math.sqrt(0.5)
    wu_r = jnp.broadcast_to((wu * s2).reshape(N_ROWS, 1), (N_ROWS, NLANE)).reshape(-1)
    wd_r = jnp.broadcast_to((wd * s2).reshape(N_ROWS, 1), (N_ROWS, NLANE)).reshape(-1)

    comb3 = _sc_combine(gab2, bias2, iud, wu_r, wd_r).reshape(H, LQ, LK)
    weights3, gout = _attention(q3, k3, v3, comb3)
    out2 = _proj(gout, W_out, b_out.reshape(1, D_MODEL))
    return out2.reshape(B, LQ, D_MODEL), weights3.reshape(B, H, LQ, LK)
